# Initial kernel scaffold; baseline (speedup 1.0000x reference)
import jax
import jax.numpy as jnp
from jax.experimental import pallas as pl


def kernel(node_features, edge_features, senders, receivers, W_fe, b_fe, W_fs, b_fs, W_fr, b_fr, W_gn, b_gn, W_gin, b_gin, W_gout, b_gout):
    raise NotImplementedError("write your pallas kernel here")


# speedup vs baseline: 1.0058x; 1.0058x over previous
import functools
import jax
import jax.numpy as jnp
from jax.experimental import pallas as pl
from jax.experimental.pallas import tpu as pltpu

_N = 10000
_E = 320000
_EBLK = 3200
_NBLK = 1000
_F32 = jnp.float32


def _edge_mm_body(ef_ref, w_ref, b_ref, out_ref):
    out_ref[...] = jnp.dot(ef_ref[...], w_ref[...],
                           preferred_element_type=_F32) + b_ref[...]


def _node_mm_body(x_ref, ws_ref, bs_ref, wr_ref, br_ref, fs_ref, fr_ref):
    x = x_ref[...]
    fs_ref[...] = jnp.dot(x, ws_ref[...], preferred_element_type=_F32) + bs_ref[...]
    fr_ref[...] = jnp.dot(x, wr_ref[...], preferred_element_type=_F32) + br_ref[...]


def _final_body(x_ref, a_ref, b_ref, fs_ref, fr_ref,
                wgn_ref, wgin_ref, wgout_ref, bias_ref, out_ref):
    agg_in = jnp.maximum(a_ref[...] + fr_ref[...], 0.0)
    agg_out = jnp.maximum(b_ref[...] + fs_ref[...], 0.0)
    out_ref[...] = (
        jnp.dot(x_ref[...], wgn_ref[...], preferred_element_type=_F32)
        + jnp.dot(agg_in, wgin_ref[...], preferred_element_type=_F32)
        + jnp.dot(agg_out, wgout_ref[...], preferred_element_type=_F32)
        + bias_ref[...])


def _edge_linear(edge_features, W_fe, b_fe):
    grid = _E // _EBLK
    return pl.pallas_call(
        _edge_mm_body,
        grid=(grid,),
        in_specs=[
            pl.BlockSpec((_EBLK, 16), lambda i: (i, 0)),
            pl.BlockSpec((16, 128), lambda i: (0, 0)),
            pl.BlockSpec((1, 128), lambda i: (0, 0)),
        ],
        out_specs=pl.BlockSpec((_EBLK, 128), lambda i: (i, 0)),
        out_shape=jax.ShapeDtypeStruct((_E, 128), _F32),
    )(edge_features, W_fe, b_fe.reshape(1, 128))


def _node_linears(x, W_fs, b_fs, W_fr, b_fr):
    grid = _N // _NBLK
    return pl.pallas_call(
        _node_mm_body,
        grid=(grid,),
        in_specs=[
            pl.BlockSpec((_NBLK, 128), lambda i: (i, 0)),
            pl.BlockSpec((128, 128), lambda i: (0, 0)),
            pl.BlockSpec((1, 128), lambda i: (0, 0)),
            pl.BlockSpec((128, 128), lambda i: (0, 0)),
            pl.BlockSpec((1, 128), lambda i: (0, 0)),
        ],
        out_specs=[
            pl.BlockSpec((_NBLK, 128), lambda i: (i, 0)),
            pl.BlockSpec((_NBLK, 128), lambda i: (i, 0)),
        ],
        out_shape=[
            jax.ShapeDtypeStruct((_N, 128), _F32),
            jax.ShapeDtypeStruct((_N, 128), _F32),
        ],
    )(x, W_fs, b_fs.reshape(1, 128), W_fr, b_fr.reshape(1, 128))


def _final(x, A, B, fs, fr, W_gn, W_gin, W_gout, bias):
    grid = _N // _NBLK
    blk = lambda i: (i, 0)
    return pl.pallas_call(
        _final_body,
        grid=(grid,),
        in_specs=[
            pl.BlockSpec((_NBLK, 128), blk),
            pl.BlockSpec((_NBLK, 128), blk),
            pl.BlockSpec((_NBLK, 128), blk),
            pl.BlockSpec((_NBLK, 128), blk),
            pl.BlockSpec((_NBLK, 128), blk),
            pl.BlockSpec((128, 128), lambda i: (0, 0)),
            pl.BlockSpec((128, 128), lambda i: (0, 0)),
            pl.BlockSpec((128, 128), lambda i: (0, 0)),
            pl.BlockSpec((1, 128), lambda i: (0, 0)),
        ],
        out_specs=pl.BlockSpec((_NBLK, 128), blk),
        out_shape=jax.ShapeDtypeStruct((_N, 128), _F32),
    )(x, A, B, fs, fr, W_gn, W_gin, W_gout, bias.reshape(1, 128))


def kernel(node_features, edge_features, senders, receivers,
           W_fe, b_fe, W_fs, b_fs, W_fr, b_fr,
           W_gn, b_gn, W_gin, b_gin, W_gout, b_gout):
    fe = _edge_linear(edge_features, W_fe, b_fe)
    fs, fr = _node_linears(node_features, W_fs, b_fs, W_fr, b_fr)
    u = fe + jnp.take(fs, senders, axis=0)
    v = fe + jnp.take(fr, receivers, axis=0)
    A = jax.ops.segment_max(u, receivers, num_segments=_N)
    B = jax.ops.segment_max(v, senders, num_segments=_N)
    bias = b_gn + b_gin + b_gout
    return _final(node_features, A, B, fs, fr, W_gn, W_gin, W_gout, bias)
